# baseline (speedup 1.0000x reference)
import functools

import jax
import jax.numpy as jnp
from jax import lax
from jax.experimental import pallas as pl
from jax.experimental.pallas import tpu as pltpu
from jax.experimental.pallas import tpu_sc as plsc

BATCH = 16384
EMBED = 64
HIDDEN = 256
NROWS = 1000000

NC = 2
NS = 16
NW = NC * NS
B_PER_W = BATCH // NW


def _sc_gather_both(utab128, itab128, uidx, iidx):
    mesh = plsc.VectorSubcoreMesh(core_axis_name="c", subcore_axis_name="s")

    @functools.partial(
        pl.kernel,
        mesh=mesh,
        out_type=[
            jax.ShapeDtypeStruct((BATCH, 128), jnp.float32),
            jax.ShapeDtypeStruct((BATCH, 128), jnp.float32),
        ],
        scratch_types=[
            pltpu.VMEM((B_PER_W,), jnp.int32),
            pltpu.VMEM((B_PER_W, 128), jnp.float32),
            pltpu.SemaphoreType.DMA,
        ],
    )
    def gather_kernel(utab_hbm, itab_hbm, uid_hbm, iid_hbm, u_out, i_out,
                      idx_v, rows_v, sem):
        wid = lax.axis_index("s") * NC + lax.axis_index("c")
        base = wid * B_PER_W
        pltpu.sync_copy(uid_hbm.at[pl.ds(base, B_PER_W)], idx_v)
        pltpu.async_copy(utab_hbm.at[idx_v], rows_v, sem).wait()
        pltpu.sync_copy(rows_v, u_out.at[pl.ds(base, B_PER_W)])
        pltpu.sync_copy(iid_hbm.at[pl.ds(base, B_PER_W)], idx_v)
        pltpu.async_copy(itab_hbm.at[idx_v], rows_v, sem).wait()
        pltpu.sync_copy(rows_v, i_out.at[pl.ds(base, B_PER_W)])

    return gather_kernel(utab128, itab128, uidx, iidx)


def _mlp_kernel(gu_ref, gi_ref, pu_ref, pi_ref, w1u_ref, w1i_ref, b1_ref,
                w2_ref, b2_ref, o_ref):
    gu = gu_ref[...]
    gi = gi_ref[...]
    u = jnp.where(pu_ref[...] == 0, gu[:, :EMBED], gu[:, EMBED:])
    i = jnp.where(pi_ref[...] == 0, gi[:, :EMBED], gi[:, EMBED:])
    h = (
        jnp.dot(u, w1u_ref[...], preferred_element_type=jnp.float32)
        + jnp.dot(i, w1i_ref[...], preferred_element_type=jnp.float32)
        + b1_ref[...]
    )
    h = jnp.where(h >= 0, h, 0.2 * h)
    out = jnp.dot(h, w2_ref[...], preferred_element_type=jnp.float32) + b2_ref[...]
    o_ref[...] = jax.nn.sigmoid(out)


def _tc_mlp(gu, gi, pu, pi, W1u, W1i, b1, W2, b2):
    blk = 2048
    grid = (BATCH // blk,)
    return pl.pallas_call(
        _mlp_kernel,
        grid=grid,
        in_specs=[
            pl.BlockSpec((blk, 128), lambda g: (g, 0)),
            pl.BlockSpec((blk, 128), lambda g: (g, 0)),
            pl.BlockSpec((blk, 1), lambda g: (g, 0)),
            pl.BlockSpec((blk, 1), lambda g: (g, 0)),
            pl.BlockSpec((EMBED, HIDDEN), lambda g: (0, 0)),
            pl.BlockSpec((EMBED, HIDDEN), lambda g: (0, 0)),
            pl.BlockSpec((1, HIDDEN), lambda g: (0, 0)),
            pl.BlockSpec((HIDDEN, 1), lambda g: (0, 0)),
            pl.BlockSpec((1, 1), lambda g: (0, 0)),
        ],
        out_specs=pl.BlockSpec((blk, 1), lambda g: (g, 0)),
        out_shape=jax.ShapeDtypeStruct((BATCH, 1), jnp.float32),
    )(gu, gi, pu, pi, W1u, W1i, b1, W2, b2)


def kernel(user_ids, item_ids, user_table, item_table, W1, b1, W2, b2):
    uid = user_ids.astype(jnp.int32)
    iid = item_ids.astype(jnp.int32)
    utab128 = user_table.reshape(NROWS // 2, 2 * EMBED)
    itab128 = item_table.reshape(NROWS // 2, 2 * EMBED)
    gu, gi = _sc_gather_both(utab128, itab128, uid // 2, iid // 2)
    pu = (uid % 2).reshape(BATCH, 1)
    pi = (iid % 2).reshape(BATCH, 1)
    W1u = W1[:EMBED]
    W1i = W1[EMBED:]
    return _tc_mlp(gu, gi, pu, pi, W1u, W1i, b1.reshape(1, HIDDEN), W2,
                   b2.reshape(1, 1))
